# baseline (speedup 1.0000x reference)
import functools

import jax
import jax.numpy as jnp
from jax import lax
from jax.experimental import pallas as pl
from jax.experimental.pallas import tpu as pltpu
from jax.experimental.pallas import tpu_sc as plsc

N_NODES = 100000
EMB = 32
L1 = 15
B = 1024
EPS = 1e-10
ROW_BLK = 2000
N_BLK = 1024

_DN = (((1,), (1,)), ((), ()))


def _sc_gather_pair(table0, idx0, table1, idx1):
    info = plsc.get_sparse_core_info()
    nw = info.num_cores * info.num_subcores
    n = idx0.shape[0]
    per = n // nw
    d = table0.shape[1]
    mesh = plsc.VectorSubcoreMesh(core_axis_name="c", subcore_axis_name="s")

    @functools.partial(
        pl.kernel, mesh=mesh,
        out_type=[jax.ShapeDtypeStruct((n, d), jnp.float32)] * 2,
        scratch_types=[pltpu.VMEM((per,), jnp.int32),
                       pltpu.VMEM((per, d), jnp.float32),
                       pltpu.SemaphoreType.DMA],
        compiler_params=pltpu.CompilerParams(use_tc_tiling_on_sc=False),
    )
    def k(t0, i0, t1, i1, o0, o1, idx_v, rows_v, sem):
        wid = lax.axis_index("s") * info.num_cores + lax.axis_index("c")
        base = wid * per
        for t, i, o in ((t0, i0, o0), (t1, i1, o1)):
            pltpu.sync_copy(i.at[pl.ds(base, per)], idx_v)
            pltpu.async_copy(t.at[idx_v], rows_v, sem).wait()
            pltpu.sync_copy(rows_v, o.at[pl.ds(base, per)])

    return k(table0, idx0, table1, idx1)


def _bn(x, n):
    m = jnp.sum(x, axis=0, keepdims=True) * (1.0 / n)
    v = jnp.sum(x * x, axis=0, keepdims=True) * (1.0 / n) - m * m
    return (x - m) * lax.rsqrt(v + EPS)


def _enc(x, w, b):
    t = jnp.tanh(lax.dot_general(_bn(x, x.shape[0]), w, _DN,
                                 preferred_element_type=jnp.float32) + b)
    return _bn(t, t.shape[0])


def _e_body(g0, g1, ew0, eb0, ew1, eb1, dw0, db0, dw1, db1,
            t00, t01, t10, t11, q0, q1):
    for g, ew, eb, dw, db, ta, tb, q in (
            (g0, ew0, eb0, dw0, db0, t00, t01, q0),
            (g1, ew1, eb1, dw1, db1, t10, t11, q1)):
        a = _enc(g[0:B, 0:L1], ew[...], eb[...])
        bb = _enc(g[B:2 * B, 0:L1], ew[...], eb[...])
        ta[...] = a
        tb[...] = bb
        o = lax.dot_general(a, dw[...], _DN,
                            preferred_element_type=jnp.float32) + db[...]
        q[...] = _bn(jnp.maximum(o, 0.0), B)


def _a1_body(e0, e1, s0, s1):
    @pl.when(pl.program_id(0) == 0)
    def _():
        s0[...] = jnp.zeros_like(s0)
        s1[...] = jnp.zeros_like(s1)
    for e, s in ((e0, s0), (e1, s1)):
        x = e[...]
        s[...] += jnp.concatenate(
            [jnp.sum(x, axis=0, keepdims=True),
             jnp.sum(x * x, axis=0, keepdims=True)], axis=0)


def _t_block(x, s, w, b):
    m = s[0:1, :] * (1.0 / N_NODES)
    v = s[1:2, :] * (1.0 / N_NODES) - m * m
    xn = (x - m) * lax.rsqrt(v + EPS)
    return jnp.tanh(lax.dot_general(xn, w, _DN,
                                    preferred_element_type=jnp.float32) + b)


def _a2_body(e0, e1, s0, s1, ew0, eb0, ew1, eb1, u0, u1):
    @pl.when(pl.program_id(0) == 0)
    def _():
        u0[...] = jnp.zeros_like(u0)
        u1[...] = jnp.zeros_like(u1)
    for e, s, ew, eb, u in ((e0, s0, ew0, eb0, u0),
                            (e1, s1, ew1, eb1, u1)):
        t = _t_block(e[...], s[...], ew[...], eb[...])
        u[...] += jnp.concatenate(
            [jnp.sum(t, axis=0, keepdims=True),
             jnp.sum(t * t, axis=0, keepdims=True)], axis=0)


def _a3_body(e0, e1, s0, s1, ew0, eb0, ew1, eb1, u0, u1, o0, o1):
    for e, s, ew, eb, u, o in ((e0, s0, ew0, eb0, u0, o0),
                               (e1, s1, ew1, eb1, u1, o1)):
        t = _t_block(e[...], s[...], ew[...], eb[...])
        m = u[0:1, :] * (1.0 / N_NODES)
        v = u[1:2, :] * (1.0 / N_NODES) - m * m
        o[...] = (t - m) * lax.rsqrt(v + EPS)


def _d_body(q, w, b, o):
    z = lax.dot_general(q[...], w[...], _DN,
                        preferred_element_type=jnp.float32) + b[...]
    o[...] = 1.0 / (1.0 + jnp.exp(-z))


def kernel(first_node, second_node, index_2_word_tensor, emb0, emb1,
           enc_w0, enc_b0, enc_w1, enc_b1, dec1_w0, dec1_b0, dec1_w1,
           dec1_b1, dec2_w0, dec2_b0, dec2_w1, dec2_b1):
    f32 = jnp.float32
    idx0 = jnp.concatenate([first_node[0], second_node[0]]).astype(jnp.int32)
    idx1 = jnp.concatenate([first_node[1], second_node[1]]).astype(jnp.int32)

    embp0 = jnp.pad(emb0, ((0, 0), (0, 1)))
    embp1 = jnp.pad(emb1, ((0, 0), (0, 1)))
    g0, g1 = _sc_gather_pair(embp0, idx0, embp1, idx1)

    eb0 = enc_b0.reshape(1, EMB)
    eb1 = enc_b1.reshape(1, EMB)
    db0 = dec1_b0.reshape(1, L1)
    db1 = dec1_b1.reshape(1, L1)

    t00, t01, t10, t11, q0, q1 = pl.pallas_call(
        _e_body,
        out_shape=[jax.ShapeDtypeStruct((B, EMB), f32)] * 4
        + [jax.ShapeDtypeStruct((B, L1), f32)] * 2,
    )(g0, g1, enc_w0, eb0, enc_w1, eb1, dec1_w0, db0, dec1_w1, db1)

    nrb = N_NODES // ROW_BLK
    eblk = pl.BlockSpec((ROW_BLK, L1), lambda i: (i, 0))
    sfull = pl.BlockSpec((2, L1), lambda i: (0, 0))
    wfull = pl.BlockSpec((EMB, L1), lambda i: (0, 0))
    bfull = pl.BlockSpec((1, EMB), lambda i: (0, 0))
    ufull = pl.BlockSpec((2, EMB), lambda i: (0, 0))

    s0, s1 = pl.pallas_call(
        _a1_body, grid=(nrb,),
        in_specs=[eblk, eblk],
        out_specs=[sfull, sfull],
        out_shape=[jax.ShapeDtypeStruct((2, L1), f32)] * 2,
    )(emb0, emb1)

    u0, u1 = pl.pallas_call(
        _a2_body, grid=(nrb,),
        in_specs=[eblk, eblk, sfull, sfull, wfull, bfull, wfull, bfull],
        out_specs=[ufull, ufull],
        out_shape=[jax.ShapeDtypeStruct((2, EMB), f32)] * 2,
    )(emb0, emb1, s0, s1, enc_w0, eb0, enc_w1, eb1)

    oblk = pl.BlockSpec((ROW_BLK, EMB), lambda i: (i, 0))
    ae0, ae1 = pl.pallas_call(
        _a3_body, grid=(nrb,),
        in_specs=[eblk, eblk, sfull, sfull, wfull, bfull, wfull, bfull,
                  ufull, ufull],
        out_specs=[oblk, oblk],
        out_shape=[jax.ShapeDtypeStruct((N_NODES, EMB), f32)] * 2,
    )(emb0, emb1, s0, s1, enc_w0, eb0, enc_w1, eb1, u0, u1)

    nnb = pl.cdiv(N_NODES, N_BLK)

    def _decode(q, w, b2):
        return pl.pallas_call(
            _d_body, grid=(nnb,),
            in_specs=[pl.BlockSpec((B, L1), lambda i: (0, 0)),
                      pl.BlockSpec((N_BLK, L1), lambda i: (i, 0)),
                      pl.BlockSpec((1, N_BLK), lambda i: (0, i))],
            out_specs=pl.BlockSpec((B, N_BLK), lambda i: (0, i)),
            out_shape=jax.ShapeDtypeStruct((B, N_NODES), f32),
        )(q, w, b2)

    o0 = _decode(q0, dec2_w0, dec2_b0.reshape(1, N_NODES))
    o1 = _decode(q1, dec2_w1, dec2_b1.reshape(1, N_NODES))

    fs0 = jnp.stack((t00, t01), axis=1)
    fs1 = jnp.stack((t11, t10), axis=1)
    return (ae0, ae1, fs0, fs1, o0, o1)


# speedup vs baseline: 1.1094x; 1.1094x over previous
import functools

import jax
import jax.numpy as jnp
from jax import lax
from jax.experimental import pallas as pl
from jax.experimental.pallas import tpu as pltpu
from jax.experimental.pallas import tpu_sc as plsc

N_NODES = 100000
EMB = 32
L1 = 15
B = 1024
EPS = 1e-10
ROW_BLK = 2000
DEC_COL = 12800
DEC_ROW = 256

_DN = (((1,), (1,)), ((), ()))


def _sc_gather_pair(table0, idx0, table1, idx1):
    info = plsc.get_sparse_core_info()
    nw = info.num_cores * info.num_subcores
    n = idx0.shape[0]
    per = n // nw
    d = table0.shape[1]
    mesh = plsc.VectorSubcoreMesh(core_axis_name="c", subcore_axis_name="s")

    @functools.partial(
        pl.kernel, mesh=mesh,
        out_type=[jax.ShapeDtypeStruct((n, d), jnp.float32)] * 2,
        scratch_types=[pltpu.VMEM((per,), jnp.int32),
                       pltpu.VMEM((per, d), jnp.float32),
                       pltpu.SemaphoreType.DMA],
        compiler_params=pltpu.CompilerParams(use_tc_tiling_on_sc=False),
    )
    def k(t0, i0, t1, i1, o0, o1, idx_v, rows_v, sem):
        wid = lax.axis_index("s") * info.num_cores + lax.axis_index("c")
        base = wid * per
        for t, i, o in ((t0, i0, o0), (t1, i1, o1)):
            pltpu.sync_copy(i.at[pl.ds(base, per)], idx_v)
            pltpu.async_copy(t.at[idx_v], rows_v, sem).wait()
            pltpu.sync_copy(rows_v, o.at[pl.ds(base, per)])

    return k(table0, idx0, table1, idx1)


def _bn(x, n):
    m = jnp.sum(x, axis=0, keepdims=True) * (1.0 / n)
    v = jnp.sum(x * x, axis=0, keepdims=True) * (1.0 / n) - m * m
    return (x - m) * lax.rsqrt(v + EPS)


def _enc(x, w, b):
    t = jnp.tanh(lax.dot_general(_bn(x, x.shape[0]), w, _DN,
                                 preferred_element_type=jnp.float32) + b)
    return _bn(t, t.shape[0])


def _e_body(g0, g1, ew0, eb0, ew1, eb1, dw0, db0, dw1, db1,
            t00, t01, t10, t11, q0, q1):
    for g, ew, eb, dw, db, ta, tb, q in (
            (g0, ew0, eb0, dw0, db0, t00, t01, q0),
            (g1, ew1, eb1, dw1, db1, t10, t11, q1)):
        a = _enc(g[0:B, 0:L1], ew[...], eb[...])
        bb = _enc(g[B:2 * B, 0:L1], ew[...], eb[...])
        ta[...] = a
        tb[...] = bb
        o = lax.dot_general(a, dw[...], _DN,
                            preferred_element_type=jnp.float32) + db[...]
        q[...] = _bn(jnp.maximum(o, 0.0), B)


def _a1_body(e0, e1, s0, s1):
    @pl.when(pl.program_id(0) == 0)
    def _():
        s0[...] = jnp.zeros_like(s0)
        s1[...] = jnp.zeros_like(s1)
    for e, s in ((e0, s0), (e1, s1)):
        x = e[...]
        s[...] += jnp.concatenate(
            [jnp.sum(x, axis=0, keepdims=True),
             jnp.sum(x * x, axis=0, keepdims=True)], axis=0)


def _t_block(x, s, w, b):
    m = s[0:1, :] * (1.0 / N_NODES)
    v = s[1:2, :] * (1.0 / N_NODES) - m * m
    xn = (x - m) * lax.rsqrt(v + EPS)
    return jnp.tanh(lax.dot_general(xn, w, _DN,
                                    preferred_element_type=jnp.float32) + b)


def _a2_body(e0, e1, s0, s1, ew0, eb0, ew1, eb1, u0, u1):
    @pl.when(pl.program_id(0) == 0)
    def _():
        u0[...] = jnp.zeros_like(u0)
        u1[...] = jnp.zeros_like(u1)
    for e, s, ew, eb, u in ((e0, s0, ew0, eb0, u0),
                            (e1, s1, ew1, eb1, u1)):
        t = _t_block(e[...], s[...], ew[...], eb[...])
        u[...] += jnp.concatenate(
            [jnp.sum(t, axis=0, keepdims=True),
             jnp.sum(t * t, axis=0, keepdims=True)], axis=0)


def _a3_body(e0, e1, s0, s1, ew0, eb0, ew1, eb1, u0, u1, o0, o1):
    for e, s, ew, eb, u, o in ((e0, s0, ew0, eb0, u0, o0),
                               (e1, s1, ew1, eb1, u1, o1)):
        t = _t_block(e[...], s[...], ew[...], eb[...])
        m = u[0:1, :] * (1.0 / N_NODES)
        v = u[1:2, :] * (1.0 / N_NODES) - m * m
        o[...] = (t - m) * lax.rsqrt(v + EPS)


def _d_body(q, w, b, o):
    z = jnp.dot(q[...], w[...], preferred_element_type=jnp.float32) + b[...]
    o[...] = 0.5 + 0.5 * jnp.tanh(0.5 * z)


def kernel(first_node, second_node, index_2_word_tensor, emb0, emb1,
           enc_w0, enc_b0, enc_w1, enc_b1, dec1_w0, dec1_b0, dec1_w1,
           dec1_b1, dec2_w0, dec2_b0, dec2_w1, dec2_b1):
    f32 = jnp.float32
    idx0 = jnp.concatenate([first_node[0], second_node[0]]).astype(jnp.int32)
    idx1 = jnp.concatenate([first_node[1], second_node[1]]).astype(jnp.int32)

    embp0 = jnp.pad(emb0, ((0, 0), (0, 1)))
    embp1 = jnp.pad(emb1, ((0, 0), (0, 1)))
    g0, g1 = _sc_gather_pair(embp0, idx0, embp1, idx1)

    eb0 = enc_b0.reshape(1, EMB)
    eb1 = enc_b1.reshape(1, EMB)
    db0 = dec1_b0.reshape(1, L1)
    db1 = dec1_b1.reshape(1, L1)

    t00, t01, t10, t11, q0, q1 = pl.pallas_call(
        _e_body,
        out_shape=[jax.ShapeDtypeStruct((B, EMB), f32)] * 4
        + [jax.ShapeDtypeStruct((B, L1), f32)] * 2,
    )(g0, g1, enc_w0, eb0, enc_w1, eb1, dec1_w0, db0, dec1_w1, db1)

    nrb = N_NODES // ROW_BLK
    eblk = pl.BlockSpec((ROW_BLK, L1), lambda i: (i, 0))
    sfull = pl.BlockSpec((2, L1), lambda i: (0, 0))
    wfull = pl.BlockSpec((EMB, L1), lambda i: (0, 0))
    bfull = pl.BlockSpec((1, EMB), lambda i: (0, 0))
    ufull = pl.BlockSpec((2, EMB), lambda i: (0, 0))

    s0, s1 = pl.pallas_call(
        _a1_body, grid=(nrb,),
        in_specs=[eblk, eblk],
        out_specs=[sfull, sfull],
        out_shape=[jax.ShapeDtypeStruct((2, L1), f32)] * 2,
    )(emb0, emb1)

    u0, u1 = pl.pallas_call(
        _a2_body, grid=(nrb,),
        in_specs=[eblk, eblk, sfull, sfull, wfull, bfull, wfull, bfull],
        out_specs=[ufull, ufull],
        out_shape=[jax.ShapeDtypeStruct((2, EMB), f32)] * 2,
    )(emb0, emb1, s0, s1, enc_w0, eb0, enc_w1, eb1)

    oblk = pl.BlockSpec((ROW_BLK, EMB), lambda i: (i, 0))
    ae0, ae1 = pl.pallas_call(
        _a3_body, grid=(nrb,),
        in_specs=[eblk, eblk, sfull, sfull, wfull, bfull, wfull, bfull,
                  ufull, ufull],
        out_specs=[oblk, oblk],
        out_shape=[jax.ShapeDtypeStruct((N_NODES, EMB), f32)] * 2,
    )(emb0, emb1, s0, s1, enc_w0, eb0, enc_w1, eb1, u0, u1)

    def _decode(q, wt, b2):
        return pl.pallas_call(
            _d_body, grid=(pl.cdiv(N_NODES, DEC_COL), B // DEC_ROW),
            in_specs=[pl.BlockSpec((DEC_ROW, L1), lambda c, r: (r, 0)),
                      pl.BlockSpec((L1, DEC_COL), lambda c, r: (0, c)),
                      pl.BlockSpec((1, DEC_COL), lambda c, r: (0, c))],
            out_specs=pl.BlockSpec((DEC_ROW, DEC_COL), lambda c, r: (r, c)),
            out_shape=jax.ShapeDtypeStruct((B, N_NODES), f32),
        )(q, wt, b2)

    o0 = _decode(q0, dec2_w0.T, dec2_b0.reshape(1, N_NODES))
    o1 = _decode(q1, dec2_w1.T, dec2_b1.reshape(1, N_NODES))

    fs0 = jnp.stack((t00, t01), axis=1)
    fs1 = jnp.stack((t11, t10), axis=1)
    return (ae0, ae1, fs0, fs1, o0, o1)
